# baseline (speedup 1.0000x reference)
import functools

import jax
import jax.numpy as jnp
from jax import lax
from jax.experimental import pallas as pl
from jax.experimental.pallas import tpu as pltpu
from jax.experimental.pallas import tpu_sc as plsc

N = 10000
D_IN = 128
D_HID = 128
D_OUT = 64
E = 320000
NW = 32
EPT = E // NW
STRIPE = N // 16

_mesh = plsc.VectorSubcoreMesh(core_axis_name="c", subcore_axis_name="s")


@functools.partial(
    pl.kernel,
    mesh=_mesh,
    out_type=jax.ShapeDtypeStruct((NW, 2, N), jnp.float32),
    compiler_params=pltpu.CompilerParams(needs_layout_passes=False),
    scratch_types=[
        pltpu.VMEM((EPT,), jnp.int32),
        pltpu.VMEM((EPT,), jnp.int32),
        pltpu.VMEM((N,), jnp.float32),
        pltpu.VMEM((N,), jnp.float32),
        pltpu.SemaphoreType.DMA,
        pltpu.SemaphoreType.DMA,
    ],
)
def _deg_kernel(src_hbm, dst_hbm, out_hbm, src_v, dst_v,
                sdeg, ddeg, semA, semB):
    cid = lax.axis_index("c")
    sid = lax.axis_index("s")
    wid = sid * 2 + cid

    a1 = pltpu.async_copy(src_hbm.at[wid], src_v, semA)
    a2 = pltpu.async_copy(dst_hbm.at[wid], dst_v, semB)

    zeros16 = jnp.zeros((16,), jnp.float32)

    def zero_body(i, _):
        sdeg[pl.ds(i * 16, 16)] = zeros16
        ddeg[pl.ds(i * 16, 16)] = zeros16
        return _

    lax.fori_loop(0, N // 16, zero_body, 0)
    a1.wait()
    a2.wait()

    ones16 = jnp.ones((16,), jnp.float32)

    def hist_body(t, _):
        for u in range(5):
            si = src_v[pl.ds(t * 80 + u * 16, 16)]
            plsc.addupdate_scatter(sdeg, [si], ones16)
            di = dst_v[pl.ds(t * 80 + u * 16, 16)]
            plsc.addupdate_scatter(ddeg, [di], ones16)
        return _

    lax.fori_loop(0, EPT // 80, hist_body, 0)

    pltpu.sync_copy(sdeg, out_hbm.at[wid, 0])
    pltpu.sync_copy(ddeg, out_hbm.at[wid, 1])


def _make_prop(D, K, CH):
    @functools.partial(
        pl.kernel,
        mesh=_mesh,
        out_type=jax.ShapeDtypeStruct((2, N, D), jnp.float32),
        compiler_params=pltpu.CompilerParams(use_tc_tiling_on_sc=False),
        scratch_types=[
            pltpu.VMEM((CH, K), jnp.int32),
            pltpu.VMEM((CH, K), jnp.int32),
            pltpu.VMEM((K, D), jnp.float32),
            pltpu.VMEM((K, D), jnp.float32),
            pltpu.VMEM_SHARED((N, D), jnp.float32),
            pltpu.SemaphoreType.DMA,
            pltpu.SemaphoreType.DMA,
            pltpu.SemaphoreType.DMA,
        ],
    )
    def prop(m_hbm, src_hbm, dst_hbm, zeros_hbm, out_hbm,
             src_v, dst_v, buf0, buf1, acc, sem0, sem1, semz):
        cid = lax.axis_index("c")
        sid = lax.axis_index("s")
        wid = sid * 2 + cid
        bufs = (buf0, buf1)
        sems = (sem0, sem1)

        def start(j, b):
            pltpu.async_copy(m_hbm.at[src_v.at[j]], bufs[b], sems[b])

        def finish(j, b):
            pltpu.make_async_copy(m_hbm.at[src_v.at[j]], bufs[b],
                                  sems[b]).wait()
            pltpu.sync_copy(bufs[b], acc.at[dst_v.at[j]], add=True)

        az = pltpu.async_copy(zeros_hbm.at[pl.ds(sid * STRIPE, STRIPE)],
                              acc.at[pl.ds(sid * STRIPE, STRIPE)], semz)
        a1 = pltpu.async_copy(src_hbm.at[wid], src_v, sems[0])
        a2 = pltpu.async_copy(dst_hbm.at[wid], dst_v, sems[1])
        a1.wait()
        a2.wait()
        start(0, 0)
        start(1, 1)
        az.wait()
        plsc.subcore_barrier()

        def body(i, _):
            j = 2 * i
            finish(j, 0)
            start(j + 2, 0)
            finish(j + 1, 1)
            start(j + 3, 1)
            return _

        lax.fori_loop(0, CH // 2 - 1, body, 0)
        finish(CH - 2, 0)
        finish(CH - 1, 1)

        plsc.subcore_barrier()
        pltpu.sync_copy(acc.at[pl.ds(sid * STRIPE, STRIPE)],
                        out_hbm.at[cid, pl.ds(sid * STRIPE, STRIPE)])

    return prop


_prop_h = _make_prop(D_HID, 100, 100)
_prop_o = _make_prop(D_OUT, 125, 80)


def _tc0_body(x_ref, w1_ref, xw_ref):
    xw_ref[...] = jnp.dot(x_ref[...], w1_ref[...],
                          preferred_element_type=jnp.float32)


def _tc1_body(deg_ref, xw_ref, m1_ref, norms_ref):
    deg = jnp.sum(deg_ref[...], axis=0)
    norms = jnp.where(deg > 0, lax.rsqrt(jnp.maximum(deg, 1.0)), 0.0)
    norms_ref[...] = norms
    m1_ref[...] = xw_ref[...] * norms[0][:, None]


def _tc2_body(p1_ref, norms_ref, b1_ref, w2_ref, m2_ref):
    agg = (p1_ref[0] + p1_ref[1]) * norms_ref[1][:, None]
    h = jnp.maximum(agg + b1_ref[...], 0.0)
    m2 = jnp.dot(h, w2_ref[...], preferred_element_type=jnp.float32)
    m2_ref[...] = m2 * norms_ref[0][:, None]


def _tc3_body(p2_ref, norms_ref, b2_ref, out_ref):
    out_ref[...] = ((p2_ref[0] + p2_ref[1]) * norms_ref[1][:, None]
                    + b2_ref[...])


_tc0 = pl.pallas_call(
    _tc0_body,
    out_shape=jax.ShapeDtypeStruct((N, D_HID), jnp.float32),
)
_tc1 = pl.pallas_call(
    _tc1_body,
    out_shape=(jax.ShapeDtypeStruct((N, D_HID), jnp.float32),
               jax.ShapeDtypeStruct((2, N), jnp.float32)),
)
_tc2 = pl.pallas_call(
    _tc2_body,
    out_shape=jax.ShapeDtypeStruct((N, D_OUT), jnp.float32),
)
_tc3 = pl.pallas_call(
    _tc3_body,
    out_shape=jax.ShapeDtypeStruct((N, D_OUT), jnp.float32),
)


@jax.jit
def _run(features, edge_index, W1, b1, W2, b2):
    src = edge_index[0].astype(jnp.int32)
    dst = edge_index[1].astype(jnp.int32)
    src_flat = src.reshape(NW, EPT)
    dst_flat = dst.reshape(NW, EPT)
    src_h = src.reshape(NW, 100, 100)
    dst_h = dst.reshape(NW, 100, 100)
    src_o = src.reshape(NW, 80, 125)
    dst_o = dst.reshape(NW, 80, 125)

    zeros_h = jnp.zeros((N, D_HID), jnp.float32)
    zeros_o = jnp.zeros((N, D_OUT), jnp.float32)
    deg_parts = _deg_kernel(src_flat, dst_flat)
    xw = _tc0(features, W1)
    m1, norms = _tc1(deg_parts, xw)
    p1 = _prop_h(m1, src_h, dst_h, zeros_h)
    m2 = _tc2(p1, norms, b1.reshape(1, D_HID), W2)
    p2 = _prop_o(m2, src_o, dst_o, zeros_o)
    out = _tc3(p2, norms, b2.reshape(1, D_OUT))
    return out


def kernel(features, edge_index, W1, b1, W2, b2):
    return _run(features, edge_index, W1, b1, W2, b2)


# speedup vs baseline: 1.0189x; 1.0189x over previous
import functools

import jax
import jax.numpy as jnp
from jax import lax
from jax.experimental import pallas as pl
from jax.experimental.pallas import tpu as pltpu
from jax.experimental.pallas import tpu_sc as plsc

N = 10000
D_IN = 128
D_HID = 128
D_OUT = 64
E = 320000
NW = 32
EPT = E // NW
STRIPE = N // 16

_mesh = plsc.VectorSubcoreMesh(core_axis_name="c", subcore_axis_name="s")


@functools.partial(
    pl.kernel,
    mesh=_mesh,
    out_type=jax.ShapeDtypeStruct((NW, 2, N), jnp.float32),
    compiler_params=pltpu.CompilerParams(needs_layout_passes=False),
    scratch_types=[
        pltpu.VMEM((EPT,), jnp.int32),
        pltpu.VMEM((EPT,), jnp.int32),
        pltpu.VMEM((N,), jnp.float32),
        pltpu.VMEM((N,), jnp.float32),
        pltpu.SemaphoreType.DMA,
        pltpu.SemaphoreType.DMA,
    ],
)
def _deg_kernel(src_hbm, dst_hbm, out_hbm, src_v, dst_v,
                sdeg, ddeg, semA, semB):
    cid = lax.axis_index("c")
    sid = lax.axis_index("s")
    wid = sid * 2 + cid

    a1 = pltpu.async_copy(src_hbm.at[wid], src_v, semA)
    a2 = pltpu.async_copy(dst_hbm.at[wid], dst_v, semB)

    zeros16 = jnp.zeros((16,), jnp.float32)

    def zero_body(i, _):
        sdeg[pl.ds(i * 16, 16)] = zeros16
        ddeg[pl.ds(i * 16, 16)] = zeros16
        return _

    lax.fori_loop(0, N // 16, zero_body, 0)
    a1.wait()
    a2.wait()

    ones16 = jnp.ones((16,), jnp.float32)

    def hist_body(t, _):
        for u in range(5):
            si = src_v[pl.ds(t * 80 + u * 16, 16)]
            plsc.addupdate_scatter(sdeg, [si], ones16)
            di = dst_v[pl.ds(t * 80 + u * 16, 16)]
            plsc.addupdate_scatter(ddeg, [di], ones16)
        return _

    lax.fori_loop(0, EPT // 80, hist_body, 0)

    pltpu.sync_copy(sdeg, out_hbm.at[wid, 0])
    pltpu.sync_copy(ddeg, out_hbm.at[wid, 1])


def _make_prop(D, K, CH):
    @functools.partial(
        pl.kernel,
        mesh=_mesh,
        out_type=jax.ShapeDtypeStruct((2, N, D), jnp.float32),
        compiler_params=pltpu.CompilerParams(use_tc_tiling_on_sc=False),
        scratch_types=[
            pltpu.VMEM((CH, K), jnp.int32),
            pltpu.VMEM((CH, K), jnp.int32),
            pltpu.VMEM((K, D), jnp.float32),
            pltpu.VMEM((K, D), jnp.float32),
            pltpu.VMEM_SHARED((N, D), jnp.float32),
            pltpu.SemaphoreType.DMA,
            pltpu.SemaphoreType.DMA,
            pltpu.SemaphoreType.DMA,
        ],
    )
    def prop(m_hbm, src_hbm, dst_hbm, zeros_hbm, out_hbm,
             src_v, dst_v, buf0, buf1, acc, sem0, sem1, semz):
        cid = lax.axis_index("c")
        sid = lax.axis_index("s")
        wid = sid * 2 + cid
        bufs = (buf0, buf1)
        sems = (sem0, sem1)

        def start(j, b):
            pltpu.async_copy(m_hbm.at[src_v.at[j]], bufs[b], sems[b])

        def finish(j, b):
            pltpu.make_async_copy(m_hbm.at[src_v.at[j]], bufs[b],
                                  sems[b]).wait()
            pltpu.sync_copy(bufs[b], acc.at[dst_v.at[j]], add=True)

        az = pltpu.async_copy(zeros_hbm.at[pl.ds(sid * STRIPE, STRIPE)],
                              acc.at[pl.ds(sid * STRIPE, STRIPE)], semz)
        a1 = pltpu.async_copy(src_hbm.at[wid], src_v, sems[0])
        a2 = pltpu.async_copy(dst_hbm.at[wid], dst_v, sems[1])
        a1.wait()
        a2.wait()
        start(0, 0)
        start(1, 1)
        az.wait()
        plsc.subcore_barrier()

        def body(i, _):
            j = 2 * i
            finish(j, 0)
            start(j + 2, 0)
            finish(j + 1, 1)
            start(j + 3, 1)
            return _

        lax.fori_loop(0, CH // 2 - 1, body, 0)
        finish(CH - 2, 0)
        finish(CH - 1, 1)

        plsc.subcore_barrier()
        pltpu.sync_copy(acc.at[pl.ds(sid * STRIPE, STRIPE)],
                        out_hbm.at[cid, pl.ds(sid * STRIPE, STRIPE)])

    return prop


_prop_h = _make_prop(D_HID, 100, 100)
_prop_o = _make_prop(D_OUT, 125, 80)


def _tc1_body(deg_ref, x_ref, w1_ref, m1_ref, norms_ref):
    deg = jnp.sum(deg_ref[...], axis=0)
    norms = jnp.where(deg > 0, lax.rsqrt(jnp.maximum(deg, 1.0)), 0.0)
    norms_ref[...] = norms
    m1 = jnp.dot(x_ref[...], w1_ref[...], preferred_element_type=jnp.float32)
    m1_ref[...] = m1 * norms[0][:, None]


def _tc2_body(p1_ref, norms_ref, b1_ref, w2_ref, m2_ref):
    agg = (p1_ref[0] + p1_ref[1]) * norms_ref[1][:, None]
    h = jnp.maximum(agg + b1_ref[...], 0.0)
    m2 = jnp.dot(h, w2_ref[...], preferred_element_type=jnp.float32)
    m2_ref[...] = m2 * norms_ref[0][:, None]


def _tc3_body(p2_ref, norms_ref, b2_ref, out_ref):
    out_ref[...] = ((p2_ref[0] + p2_ref[1]) * norms_ref[1][:, None]
                    + b2_ref[...])


_tc1 = pl.pallas_call(
    _tc1_body,
    out_shape=(jax.ShapeDtypeStruct((N, D_HID), jnp.float32),
               jax.ShapeDtypeStruct((2, N), jnp.float32)),
)
_tc2 = pl.pallas_call(
    _tc2_body,
    out_shape=jax.ShapeDtypeStruct((N, D_OUT), jnp.float32),
)
_tc3 = pl.pallas_call(
    _tc3_body,
    out_shape=jax.ShapeDtypeStruct((N, D_OUT), jnp.float32),
)


@jax.jit
def _run(features, edge_index, W1, b1, W2, b2):
    src = edge_index[0].astype(jnp.int32)
    dst = edge_index[1].astype(jnp.int32)
    src_flat = src.reshape(NW, EPT)
    dst_flat = dst.reshape(NW, EPT)
    src_h = src.reshape(NW, 100, 100)
    dst_h = dst.reshape(NW, 100, 100)
    src_o = src.reshape(NW, 80, 125)
    dst_o = dst.reshape(NW, 80, 125)

    zeros_h = jnp.zeros((N, D_HID), jnp.float32)
    zeros_o = jnp.zeros((N, D_OUT), jnp.float32)
    deg_parts = _deg_kernel(src_flat, dst_flat)
    m1, norms = _tc1(deg_parts, features, W1)
    p1 = _prop_h(m1, src_h, dst_h, zeros_h)
    m2 = _tc2(p1, norms, b1.reshape(1, D_HID), W2)
    p2 = _prop_o(m2, src_o, dst_o, zeros_o)
    out = _tc3(p2, norms, b2.reshape(1, D_OUT))
    return out


def kernel(features, edge_index, W1, b1, W2, b2):
    return _run(features, edge_index, W1, b1, W2, b2)


# speedup vs baseline: 1.0751x; 1.0551x over previous
import functools

import jax
import jax.numpy as jnp
from jax import lax
from jax.experimental import pallas as pl
from jax.experimental.pallas import tpu as pltpu
from jax.experimental.pallas import tpu_sc as plsc

N = 10000
D_IN = 128
D_HID = 128
D_OUT = 64
E = 320000
NW = 32
EPT = E // NW
STRIPE = N // 16

_mesh = plsc.VectorSubcoreMesh(core_axis_name="c", subcore_axis_name="s")


@functools.partial(
    pl.kernel,
    mesh=_mesh,
    out_type=jax.ShapeDtypeStruct((NW, 2, N), jnp.float32),
    compiler_params=pltpu.CompilerParams(needs_layout_passes=False),
    scratch_types=[
        pltpu.VMEM((EPT,), jnp.int32),
        pltpu.VMEM((EPT,), jnp.int32),
        pltpu.VMEM((N,), jnp.float32),
        pltpu.VMEM((N,), jnp.float32),
        pltpu.SemaphoreType.DMA,
        pltpu.SemaphoreType.DMA,
    ],
)
def _deg_kernel(src_hbm, dst_hbm, out_hbm, src_v, dst_v,
                sdeg, ddeg, semA, semB):
    cid = lax.axis_index("c")
    sid = lax.axis_index("s")
    wid = sid * 2 + cid

    a1 = pltpu.async_copy(src_hbm.at[wid], src_v, semA)
    a2 = pltpu.async_copy(dst_hbm.at[wid], dst_v, semB)

    zeros16 = jnp.zeros((16,), jnp.float32)

    def zero_body(i, _):
        sdeg[pl.ds(i * 16, 16)] = zeros16
        ddeg[pl.ds(i * 16, 16)] = zeros16
        return _

    lax.fori_loop(0, N // 16, zero_body, 0)
    a1.wait()
    a2.wait()

    ones16 = jnp.ones((16,), jnp.float32)

    def hist_body(t, _):
        for u in range(5):
            si = src_v[pl.ds(t * 80 + u * 16, 16)]
            plsc.addupdate_scatter(sdeg, [si], ones16)
            di = dst_v[pl.ds(t * 80 + u * 16, 16)]
            plsc.addupdate_scatter(ddeg, [di], ones16)
        return _

    lax.fori_loop(0, EPT // 80, hist_body, 0)

    pltpu.sync_copy(sdeg, out_hbm.at[wid, 0])
    pltpu.sync_copy(ddeg, out_hbm.at[wid, 1])


def _make_prop(D, K, CH, NB):
    @functools.partial(
        pl.kernel,
        mesh=_mesh,
        out_type=jax.ShapeDtypeStruct((2, N, D), jnp.float32),
        compiler_params=pltpu.CompilerParams(use_tc_tiling_on_sc=False),
        scratch_types=[
            pltpu.VMEM((CH, K), jnp.int32),
            pltpu.VMEM((CH, K), jnp.int32),
            [pltpu.VMEM((K, D), jnp.float32)] * NB,
            pltpu.VMEM_SHARED((N, D), jnp.float32),
            [pltpu.SemaphoreType.DMA] * NB,
            pltpu.SemaphoreType.DMA,
        ],
    )
    def prop(m_hbm, src_hbm, dst_hbm, zeros_hbm, out_hbm,
             src_v, dst_v, bufs, acc, sems, semz):
        cid = lax.axis_index("c")
        sid = lax.axis_index("s")
        wid = sid * 2 + cid

        def start(j, b):
            pltpu.async_copy(m_hbm.at[src_v.at[j]], bufs[b], sems[b])

        def finish(j, b):
            pltpu.make_async_copy(m_hbm.at[src_v.at[j]], bufs[b],
                                  sems[b]).wait()
            pltpu.sync_copy(bufs[b], acc.at[dst_v.at[j]], add=True)

        az = pltpu.async_copy(zeros_hbm.at[pl.ds(sid * STRIPE, STRIPE)],
                              acc.at[pl.ds(sid * STRIPE, STRIPE)], semz)
        a1 = pltpu.async_copy(src_hbm.at[wid], src_v, sems[0])
        a2 = pltpu.async_copy(dst_hbm.at[wid], dst_v, sems[1 % NB])
        a1.wait()
        a2.wait()
        for j in range(NB):
            start(j, j)
        az.wait()
        plsc.subcore_barrier()

        n_full = (CH - NB) // NB

        def body(i, _):
            j0 = NB * i
            for b in range(NB):
                finish(j0 + b, b)
                start(j0 + b + NB, b)
            return _

        lax.fori_loop(0, n_full, body, 0)
        for j in range(n_full * NB, CH):
            finish(j, j % NB)
            if j + NB < CH:
                start(j + NB, j % NB)

        plsc.subcore_barrier()
        pltpu.sync_copy(acc.at[pl.ds(sid * STRIPE, STRIPE)],
                        out_hbm.at[cid, pl.ds(sid * STRIPE, STRIPE)])

    return prop


_prop_h = _make_prop(D_HID, 100, 100, 2)
_prop_o = _make_prop(D_OUT, 125, 80, 3)


def _tc1_body(deg_ref, x_ref, w1_ref, m1_ref, norms_ref):
    deg = jnp.sum(deg_ref[...], axis=0)
    norms = jnp.where(deg > 0, lax.rsqrt(jnp.maximum(deg, 1.0)), 0.0)
    norms_ref[...] = norms
    m1 = jnp.dot(x_ref[...], w1_ref[...], preferred_element_type=jnp.float32)
    m1_ref[...] = m1 * norms[0][:, None]


def _tc2_body(p1_ref, norms_ref, b1_ref, w2_ref, m2_ref):
    agg = (p1_ref[0] + p1_ref[1]) * norms_ref[1][:, None]
    h = jnp.maximum(agg + b1_ref[...], 0.0)
    m2 = jnp.dot(h, w2_ref[...], preferred_element_type=jnp.float32)
    m2_ref[...] = m2 * norms_ref[0][:, None]


def _tc3_body(p2_ref, norms_ref, b2_ref, out_ref):
    out_ref[...] = ((p2_ref[0] + p2_ref[1]) * norms_ref[1][:, None]
                    + b2_ref[...])


_tc1 = pl.pallas_call(
    _tc1_body,
    out_shape=(jax.ShapeDtypeStruct((N, D_HID), jnp.float32),
               jax.ShapeDtypeStruct((2, N), jnp.float32)),
)
_tc2 = pl.pallas_call(
    _tc2_body,
    out_shape=jax.ShapeDtypeStruct((N, D_OUT), jnp.float32),
)
_tc3 = pl.pallas_call(
    _tc3_body,
    out_shape=jax.ShapeDtypeStruct((N, D_OUT), jnp.float32),
)


@jax.jit
def _run(features, edge_index, W1, b1, W2, b2):
    src = edge_index[0].astype(jnp.int32)
    dst = edge_index[1].astype(jnp.int32)
    src_flat = src.reshape(NW, EPT)
    dst_flat = dst.reshape(NW, EPT)
    src_h = src.reshape(NW, 100, 100)
    dst_h = dst.reshape(NW, 100, 100)
    src_o = src.reshape(NW, 80, 125)
    dst_o = dst.reshape(NW, 80, 125)

    zeros_h = jnp.zeros((N, D_HID), jnp.float32)
    zeros_o = jnp.zeros((N, D_OUT), jnp.float32)
    deg_parts = _deg_kernel(src_flat, dst_flat)
    m1, norms = _tc1(deg_parts, features, W1)
    p1 = _prop_h(m1, src_h, dst_h, zeros_h)
    m2 = _tc2(p1, norms, b1.reshape(1, D_HID), W2)
    p2 = _prop_o(m2, src_o, dst_o, zeros_o)
    out = _tc3(p2, norms, b2.reshape(1, D_OUT))
    return out


def kernel(features, edge_index, W1, b1, W2, b2):
    return _run(features, edge_index, W1, b1, W2, b2)


# speedup vs baseline: 1.1434x; 1.0635x over previous
import functools

import jax
import jax.numpy as jnp
from jax import lax
from jax.experimental import pallas as pl
from jax.experimental.pallas import tpu as pltpu
from jax.experimental.pallas import tpu_sc as plsc

N = 10000
D_IN = 128
D_HID = 128
D_OUT = 64
E = 320000
NW = 32
EPT = E // NW
STRIPE = N // 16

_mesh = plsc.VectorSubcoreMesh(core_axis_name="c", subcore_axis_name="s")


@functools.partial(
    pl.kernel,
    mesh=_mesh,
    out_type=jax.ShapeDtypeStruct((NW, 2, N), jnp.float32),
    compiler_params=pltpu.CompilerParams(needs_layout_passes=False),
    scratch_types=[
        pltpu.VMEM((EPT,), jnp.int32),
        pltpu.VMEM((EPT,), jnp.int32),
        pltpu.VMEM((N,), jnp.float32),
        pltpu.VMEM((N,), jnp.float32),
        pltpu.SemaphoreType.DMA,
        pltpu.SemaphoreType.DMA,
    ],
)
def _deg_kernel(src_hbm, dst_hbm, out_hbm, src_v, dst_v,
                sdeg, ddeg, semA, semB):
    cid = lax.axis_index("c")
    sid = lax.axis_index("s")
    wid = sid * 2 + cid

    a1 = pltpu.async_copy(src_hbm.at[wid], src_v, semA)
    a2 = pltpu.async_copy(dst_hbm.at[wid], dst_v, semB)

    zeros16 = jnp.zeros((16,), jnp.float32)

    def zero_body(i, _):
        sdeg[pl.ds(i * 16, 16)] = zeros16
        ddeg[pl.ds(i * 16, 16)] = zeros16
        return _

    lax.fori_loop(0, N // 16, zero_body, 0)
    a1.wait()
    a2.wait()

    ones16 = jnp.ones((16,), jnp.float32)

    def hist_body(t, _):
        for u in range(5):
            si = src_v[pl.ds(t * 80 + u * 16, 16)]
            plsc.addupdate_scatter(sdeg, [si], ones16)
            di = dst_v[pl.ds(t * 80 + u * 16, 16)]
            plsc.addupdate_scatter(ddeg, [di], ones16)
        return _

    lax.fori_loop(0, EPT // 80, hist_body, 0)

    pltpu.sync_copy(sdeg, out_hbm.at[wid, 0])
    pltpu.sync_copy(ddeg, out_hbm.at[wid, 1])


def _make_prop(D, K, CH, NB):
    @functools.partial(
        pl.kernel,
        mesh=_mesh,
        out_type=jax.ShapeDtypeStruct((2, N, D), jnp.float32),
        compiler_params=pltpu.CompilerParams(use_tc_tiling_on_sc=False),
        scratch_types=[
            pltpu.VMEM((CH, K), jnp.int32),
            pltpu.VMEM((CH, K), jnp.int32),
            [pltpu.VMEM((K, D), jnp.float32)] * NB,
            pltpu.VMEM_SHARED((N, D), jnp.float32),
            [pltpu.SemaphoreType.DMA] * NB,
            pltpu.SemaphoreType.DMA,
        ],
    )
    def prop(m_hbm, src_hbm, dst_hbm, zeros_hbm, out_hbm,
             src_v, dst_v, bufs, acc, sems, semz):
        cid = lax.axis_index("c")
        sid = lax.axis_index("s")
        wid = sid * 2 + cid

        def start(j, b):
            pltpu.async_copy(m_hbm.at[src_v.at[j]], bufs[b], sems[b])

        def finish(j, b):
            pltpu.make_async_copy(m_hbm.at[src_v.at[j]], bufs[b],
                                  sems[b]).wait()
            pltpu.sync_copy(bufs[b], acc.at[dst_v.at[j]], add=True)

        az = pltpu.async_copy(zeros_hbm.at[pl.ds(sid * STRIPE, STRIPE)],
                              acc.at[pl.ds(sid * STRIPE, STRIPE)], semz)
        a1 = pltpu.async_copy(src_hbm.at[wid], src_v, sems[0])
        a2 = pltpu.async_copy(dst_hbm.at[wid], dst_v, sems[1 % NB])
        a1.wait()
        a2.wait()
        for j in range(NB):
            start(j, j)
        az.wait()
        plsc.subcore_barrier()

        n_full = (CH - NB) // NB

        def body(i, _):
            j0 = NB * i
            for b in range(NB):
                finish(j0 + b, b)
                start(j0 + b + NB, b)
            return _

        lax.fori_loop(0, n_full, body, 0)
        for j in range(n_full * NB, CH):
            finish(j, j % NB)
            if j + NB < CH:
                start(j + NB, j % NB)

        plsc.subcore_barrier()
        pltpu.sync_copy(acc.at[pl.ds(sid * STRIPE, STRIPE)],
                        out_hbm.at[cid, pl.ds(sid * STRIPE, STRIPE)])

    return prop


_prop_h = _make_prop(D_HID, 80, 125, 3)
_prop_o = _make_prop(D_OUT, 125, 80, 3)


def _tc1_body(deg_ref, x_ref, w1_ref, m1_ref, norms_ref):
    deg = jnp.sum(deg_ref[...], axis=0)
    norms = jnp.where(deg > 0, lax.rsqrt(jnp.maximum(deg, 1.0)), 0.0)
    norms_ref[...] = norms
    m1 = jnp.dot(x_ref[...], w1_ref[...], preferred_element_type=jnp.float32)
    m1_ref[...] = m1 * norms[0][:, None]


def _tc2_body(p1_ref, norms_ref, b1_ref, w2_ref, m2_ref):
    agg = (p1_ref[0] + p1_ref[1]) * norms_ref[1][:, None]
    h = jnp.maximum(agg + b1_ref[...], 0.0)
    m2 = jnp.dot(h, w2_ref[...], preferred_element_type=jnp.float32)
    m2_ref[...] = m2 * norms_ref[0][:, None]


def _tc3_body(p2_ref, norms_ref, b2_ref, out_ref):
    out_ref[...] = ((p2_ref[0] + p2_ref[1]) * norms_ref[1][:, None]
                    + b2_ref[...])


_tc1 = pl.pallas_call(
    _tc1_body,
    out_shape=(jax.ShapeDtypeStruct((N, D_HID), jnp.float32),
               jax.ShapeDtypeStruct((2, N), jnp.float32)),
)
_tc2 = pl.pallas_call(
    _tc2_body,
    out_shape=jax.ShapeDtypeStruct((N, D_OUT), jnp.float32),
)
_tc3 = pl.pallas_call(
    _tc3_body,
    out_shape=jax.ShapeDtypeStruct((N, D_OUT), jnp.float32),
)


@jax.jit
def _run(features, edge_index, W1, b1, W2, b2):
    src = edge_index[0].astype(jnp.int32)
    dst = edge_index[1].astype(jnp.int32)
    src_flat = src.reshape(NW, EPT)
    dst_flat = dst.reshape(NW, EPT)
    src_h = src.reshape(NW, 125, 80)
    dst_h = dst.reshape(NW, 125, 80)
    src_o = src.reshape(NW, 80, 125)
    dst_o = dst.reshape(NW, 80, 125)

    zeros_h = jnp.zeros((N, D_HID), jnp.float32)
    zeros_o = jnp.zeros((N, D_OUT), jnp.float32)
    deg_parts = _deg_kernel(src_flat, dst_flat)
    m1, norms = _tc1(deg_parts, features, W1)
    p1 = _prop_h(m1, src_h, dst_h, zeros_h)
    m2 = _tc2(p1, norms, b1.reshape(1, D_HID), W2)
    p2 = _prop_o(m2, src_o, dst_o, zeros_o)
    out = _tc3(p2, norms, b2.reshape(1, D_OUT))
    return out


def kernel(features, edge_index, W1, b1, W2, b2):
    return _run(features, edge_index, W1, b1, W2, b2)


# speedup vs baseline: 1.1550x; 1.0102x over previous
import functools

import jax
import jax.numpy as jnp
from jax import lax
from jax.experimental import pallas as pl
from jax.experimental.pallas import tpu as pltpu
from jax.experimental.pallas import tpu_sc as plsc

N = 10000
D_IN = 128
D_HID = 128
D_OUT = 64
E = 320000
NW = 32
EPT = E // NW
STRIPE = N // 16

_mesh = plsc.VectorSubcoreMesh(core_axis_name="c", subcore_axis_name="s")


@functools.partial(
    pl.kernel,
    mesh=_mesh,
    out_type=jax.ShapeDtypeStruct((NW, 2, N), jnp.float32),
    compiler_params=pltpu.CompilerParams(needs_layout_passes=False),
    scratch_types=[
        pltpu.VMEM((EPT,), jnp.int32),
        pltpu.VMEM((EPT,), jnp.int32),
        pltpu.VMEM((N,), jnp.float32),
        pltpu.VMEM((N,), jnp.float32),
        pltpu.SemaphoreType.DMA,
        pltpu.SemaphoreType.DMA,
    ],
)
def _deg_kernel(src_hbm, dst_hbm, out_hbm, src_v, dst_v,
                sdeg, ddeg, semA, semB):
    cid = lax.axis_index("c")
    sid = lax.axis_index("s")
    wid = sid * 2 + cid

    a1 = pltpu.async_copy(src_hbm.at[wid], src_v, semA)
    a2 = pltpu.async_copy(dst_hbm.at[wid], dst_v, semB)

    zeros16 = jnp.zeros((16,), jnp.float32)

    def zero_body(i, _):
        sdeg[pl.ds(i * 16, 16)] = zeros16
        ddeg[pl.ds(i * 16, 16)] = zeros16
        return _

    lax.fori_loop(0, N // 16, zero_body, 0)
    a1.wait()
    a2.wait()

    ones16 = jnp.ones((16,), jnp.float32)

    def hist_body(t, _):
        for u in range(5):
            si = src_v[pl.ds(t * 80 + u * 16, 16)]
            plsc.addupdate_scatter(sdeg, [si], ones16)
            di = dst_v[pl.ds(t * 80 + u * 16, 16)]
            plsc.addupdate_scatter(ddeg, [di], ones16)
        return _

    lax.fori_loop(0, EPT // 80, hist_body, 0)

    pltpu.sync_copy(sdeg, out_hbm.at[wid, 0])
    pltpu.sync_copy(ddeg, out_hbm.at[wid, 1])


def _make_prop(D, K, CH, NB):
    @functools.partial(
        pl.kernel,
        mesh=_mesh,
        out_type=jax.ShapeDtypeStruct((2, N, D), jnp.float32),
        compiler_params=pltpu.CompilerParams(use_tc_tiling_on_sc=False),
        scratch_types=[
            pltpu.VMEM((CH, K), jnp.int32),
            pltpu.VMEM((CH, K), jnp.int32),
            [pltpu.VMEM((K, D), jnp.float32)] * NB,
            pltpu.VMEM_SHARED((N, D), jnp.float32),
            [pltpu.SemaphoreType.DMA] * NB,
            pltpu.SemaphoreType.DMA,
        ],
    )
    def prop(m_hbm, src_hbm, dst_hbm, zeros_hbm, out_hbm,
             src_v, dst_v, bufs, acc, sems, semz):
        cid = lax.axis_index("c")
        sid = lax.axis_index("s")
        wid = sid * 2 + cid

        def start(j, b):
            pltpu.async_copy(m_hbm.at[src_v.at[j]], bufs[b], sems[b])

        def finish(j, b):
            pltpu.make_async_copy(m_hbm.at[src_v.at[j]], bufs[b],
                                  sems[b]).wait()
            pltpu.sync_copy(bufs[b], acc.at[dst_v.at[j]], add=True)

        az = pltpu.async_copy(zeros_hbm.at[pl.ds(sid * STRIPE, STRIPE)],
                              acc.at[pl.ds(sid * STRIPE, STRIPE)], semz)
        a1 = pltpu.async_copy(src_hbm.at[wid], src_v, sems[0])
        a2 = pltpu.async_copy(dst_hbm.at[wid], dst_v, sems[1 % NB])
        a1.wait()
        a2.wait()
        for j in range(NB):
            start(j, j)
        az.wait()
        plsc.subcore_barrier()

        n_full = (CH - NB) // NB

        def body(i, _):
            j0 = NB * i
            for b in range(NB):
                finish(j0 + b, b)
                start(j0 + b + NB, b)
            return _

        lax.fori_loop(0, n_full, body, 0)
        for j in range(n_full * NB, CH):
            finish(j, j % NB)
            if j + NB < CH:
                start(j + NB, j % NB)

        plsc.subcore_barrier()
        pltpu.sync_copy(acc.at[pl.ds(sid * STRIPE, STRIPE)],
                        out_hbm.at[cid, pl.ds(sid * STRIPE, STRIPE)])

    return prop


_prop_h = _make_prop(D_HID, 80, 125, 3)
_prop_o = _make_prop(D_OUT, 125, 80, 4)


def _tc1_body(deg_ref, x_ref, w1_ref, m1_ref, norms_ref):
    deg = jnp.sum(deg_ref[...], axis=0)
    norms = jnp.where(deg > 0, lax.rsqrt(jnp.maximum(deg, 1.0)), 0.0)
    norms_ref[...] = norms
    m1 = jnp.dot(x_ref[...], w1_ref[...], preferred_element_type=jnp.float32)
    m1_ref[...] = m1 * norms[0][:, None]


def _tc2_body(p1_ref, norms_ref, b1_ref, w2_ref, m2_ref):
    agg = (p1_ref[0] + p1_ref[1]) * norms_ref[1][:, None]
    h = jnp.maximum(agg + b1_ref[...], 0.0)
    m2 = jnp.dot(h, w2_ref[...], preferred_element_type=jnp.float32)
    m2_ref[...] = m2 * norms_ref[0][:, None]


def _tc3_body(p2_ref, norms_ref, b2_ref, out_ref):
    out_ref[...] = ((p2_ref[0] + p2_ref[1]) * norms_ref[1][:, None]
                    + b2_ref[...])


_tc1 = pl.pallas_call(
    _tc1_body,
    out_shape=(jax.ShapeDtypeStruct((N, D_HID), jnp.float32),
               jax.ShapeDtypeStruct((2, N), jnp.float32)),
)
_tc2 = pl.pallas_call(
    _tc2_body,
    out_shape=jax.ShapeDtypeStruct((N, D_OUT), jnp.float32),
)
_tc3 = pl.pallas_call(
    _tc3_body,
    out_shape=jax.ShapeDtypeStruct((N, D_OUT), jnp.float32),
)


@jax.jit
def _run(features, edge_index, W1, b1, W2, b2):
    src = edge_index[0].astype(jnp.int32)
    dst = edge_index[1].astype(jnp.int32)
    src_flat = src.reshape(NW, EPT)
    dst_flat = dst.reshape(NW, EPT)
    src_h = src.reshape(NW, 125, 80)
    dst_h = dst.reshape(NW, 125, 80)
    src_o = src.reshape(NW, 80, 125)
    dst_o = dst.reshape(NW, 80, 125)

    zeros_h = jnp.zeros((N, D_HID), jnp.float32)
    zeros_o = jnp.zeros((N, D_OUT), jnp.float32)
    deg_parts = _deg_kernel(src_flat, dst_flat)
    m1, norms = _tc1(deg_parts, features, W1)
    p1 = _prop_h(m1, src_h, dst_h, zeros_h)
    m2 = _tc2(p1, norms, b1.reshape(1, D_HID), W2)
    p2 = _prop_o(m2, src_o, dst_o, zeros_o)
    out = _tc3(p2, norms, b2.reshape(1, D_OUT))
    return out


def kernel(features, edge_index, W1, b1, W2, b2):
    return _run(features, edge_index, W1, b1, W2, b2)
